# baseline (speedup 1.0000x reference)
import functools

import jax
import jax.numpy as jnp
import numpy as np
from jax import lax
from jax.experimental import pallas as pl
from jax.experimental.pallas import tpu as pltpu
from jax.experimental.pallas import tpu_sc as plsc

_STRENGTH = 0.1
_NEG_INF = float("-inf")
_INT_MIN = np.int32(-(2 ** 31))
_INT_MAX = np.int32(2 ** 31 - 1)
_LOW_MASK = np.int32(16383)
_HIGH_MASK = np.int32(-16384)

_NC = 2
_NS = 16
_LANES = 16
_KPAD = 16
_CHUNK = 16


def _proj_norm_kernel(feat_ref, wt_ref, proj_ref, hi_ref, lo_ref):
    proj = jnp.dot(feat_ref[...], wt_ref[...],
                   preferred_element_type=jnp.float32,
                   precision=jax.lax.Precision.HIGHEST)
    proj_ref[...] = proj
    norm = jnp.sqrt(jnp.sum(proj * proj, axis=1, keepdims=True))
    normed = proj / jnp.maximum(norm, 1e-12)
    hi = normed.astype(jnp.bfloat16)
    hi_ref[...] = hi
    lo_ref[...] = (normed - hi.astype(jnp.float32)).astype(jnp.bfloat16)


def _f32_to_ikey(x):
    bits = jax.lax.bitcast_convert_type(x, jnp.int32)
    return bits ^ (jax.lax.shift_right_arithmetic(bits, 31) & _INT_MAX)


def _ikey_to_f32(k):
    bits = k ^ (jax.lax.shift_right_arithmetic(k, 31) & _INT_MAX)
    return jax.lax.bitcast_convert_type(bits, jnp.float32)


def _topk_kernel(rhi_ref, rlo_ref, thi_ref, tlo_ref, w_ref, id_ref, keys_ref,
                 *, block_rows, n, kp1):
    pid = pl.program_id(0)
    row0 = pid * block_rows

    dot = functools.partial(jnp.dot, preferred_element_type=jnp.float32,
                            precision=jax.lax.Precision.DEFAULT)
    sims = (dot(rhi_ref[...], thi_ref[...])
            + dot(rhi_ref[...], tlo_ref[...])
            + dot(rlo_ref[...], thi_ref[...]))
    col_iota = jax.lax.broadcasted_iota(jnp.int32, (block_rows, n), 1)
    keys_ref[...] = (_f32_to_ikey(sims) & _HIGH_MASK) | (_LOW_MASK - col_iota)

    vals = []
    idxs = []
    m_prev = jnp.full((block_rows, 1), _INT_MAX, jnp.int32)
    for _ in range(kp1):
        cand = jnp.where(keys_ref[...] < m_prev, keys_ref[...], _INT_MIN)
        m = jnp.max(cand, axis=1, keepdims=True)
        idxs.append(_LOW_MASK - (m & _LOW_MASK))
        vals.append(_ikey_to_f32(m & _HIGH_MASK))
        m_prev = m

    row_ids = row0 + jax.lax.broadcasted_iota(jnp.int32, (block_rows, 1), 0)

    valid = [i != row_ids for i in idxs]
    mmax = functools.reduce(
        jnp.maximum,
        [jnp.where(v, x, _NEG_INF) for v, x in zip(valid, vals)])
    exps = [jnp.where(v, jnp.exp(x - mmax), 0.0)
            for v, x in zip(valid, vals)]
    denom = functools.reduce(jnp.add, exps)
    weights = [e / denom for e in exps]
    ids = [jnp.where(v, i, 0) for v, i in zip(valid, idxs)]

    zero_i = jnp.zeros((block_rows, 1), jnp.int32)
    id_ref[...] = jnp.concatenate(
        ids + [zero_i] * (_KPAD - kp1), axis=1)
    zero_wb = jnp.zeros((block_rows, 1, _LANES), jnp.float32)
    wb = jnp.concatenate(
        [jnp.broadcast_to(w.reshape(block_rows, 1, 1),
                          (block_rows, 1, _LANES)) for w in weights]
        + [zero_wb] * (_KPAD - kp1), axis=1)
    w_ref[...] = wb.reshape(block_rows * _KPAD, _LANES)


def _sc_gather_kernel(proj_hbm, idx_hbm, w_hbm, feat_hbm, emb_hbm, out_hbm,
                      idx_v, w_v, rows_v, feat_v, out_v, emb_v, sem,
                      *, rows_per_worker):
    wid = lax.axis_index("s") * _NC + lax.axis_index("c")
    nchunks = rows_per_worker // _CHUNK
    nd = 128 // _LANES
    pltpu.sync_copy(emb_hbm, emb_v)

    def chunk_body(c, _):
        row0 = wid * rows_per_worker + c * _CHUNK
        flat0 = row0 * _KPAD
        pltpu.sync_copy(idx_hbm.at[pl.ds(flat0, _CHUNK * _KPAD)], idx_v)
        gather = pltpu.async_copy(proj_hbm.at[idx_v], rows_v, sem)
        pltpu.sync_copy(w_hbm.at[pl.ds(flat0, _CHUNK * _KPAD)], w_v)
        pltpu.sync_copy(feat_hbm.at[pl.ds(row0, _CHUNK)], feat_v)
        gather.wait()

        def row_body(r, _):
            for d in range(nd):
                dsl = pl.ds(d * _LANES, _LANES)
                acc = jnp.zeros((_LANES,), jnp.float32)
                for k in range(_KPAD):
                    j = r * _KPAD + k
                    acc = acc + w_v[j, :] * rows_v[j, dsl]
                out_v[r, dsl] = (feat_v[r, dsl]
                                 + _STRENGTH * (acc + emb_v[dsl]))
            return 0

        lax.fori_loop(0, _CHUNK, row_body, 0)
        pltpu.sync_copy(out_v, out_hbm.at[pl.ds(row0, _CHUNK)])
        return 0

    lax.fori_loop(0, nchunks, chunk_body, 0)


def kernel(feats, W, emb):
    feat = feats[0]
    n, d = feat.shape
    k = min(10, n // 10)
    kp1 = k + 1

    block_rows = 400 if n % 400 == 0 else 200

    proj, normed_hi, normed_lo = pl.pallas_call(
        _proj_norm_kernel,
        grid=(n // block_rows,),
        in_specs=[
            pl.BlockSpec((block_rows, d), lambda i: (i, 0)),
            pl.BlockSpec((d, d), lambda i: (0, 0)),
        ],
        out_specs=[
            pl.BlockSpec((block_rows, d), lambda i: (i, 0)),
            pl.BlockSpec((block_rows, d), lambda i: (i, 0)),
            pl.BlockSpec((block_rows, d), lambda i: (i, 0)),
        ],
        out_shape=[
            jax.ShapeDtypeStruct((n, d), jnp.float32),
            jax.ShapeDtypeStruct((n, d), jnp.bfloat16),
            jax.ShapeDtypeStruct((n, d), jnp.bfloat16),
        ],
    )(feat, W.T)

    w16, id16 = pl.pallas_call(
        functools.partial(_topk_kernel, block_rows=block_rows, n=n, kp1=kp1),
        grid=(n // block_rows,),
        in_specs=[
            pl.BlockSpec((block_rows, d), lambda i: (i, 0)),
            pl.BlockSpec((block_rows, d), lambda i: (i, 0)),
            pl.BlockSpec((d, n), lambda i: (0, 0)),
            pl.BlockSpec((d, n), lambda i: (0, 0)),
        ],
        out_specs=[
            pl.BlockSpec((block_rows * _KPAD, _LANES), lambda i: (i, 0)),
            pl.BlockSpec((block_rows, _KPAD), lambda i: (i, 0)),
        ],
        out_shape=[
            jax.ShapeDtypeStruct((n * _KPAD, _LANES), jnp.float32),
            jax.ShapeDtypeStruct((n, _KPAD), jnp.int32),
        ],
        scratch_shapes=[pltpu.VMEM((block_rows, n), jnp.int32)],
    )(normed_hi, normed_lo, normed_hi.T, normed_lo.T)

    nw = _NC * _NS
    rows_per_worker = -(-n // (nw * _CHUNK)) * _CHUNK
    npad = rows_per_worker * nw
    pad = npad - n
    idx_flat = jnp.pad(id16, ((0, pad), (0, 0))).reshape(-1)
    w_bcast = jnp.pad(w16, ((0, pad * _KPAD), (0, 0)))
    feat_pad = jnp.pad(feat, ((0, pad), (0, 0)))

    mesh = plsc.VectorSubcoreMesh(core_axis_name="c", subcore_axis_name="s")
    sc = pl.kernel(
        functools.partial(_sc_gather_kernel, rows_per_worker=rows_per_worker),
        mesh=mesh,
        out_type=jax.ShapeDtypeStruct((npad, d), jnp.float32),
        scratch_types=[
            pltpu.VMEM((_CHUNK * _KPAD,), jnp.int32),
            pltpu.VMEM((_CHUNK * _KPAD, _LANES), jnp.float32),
            pltpu.VMEM((_CHUNK * _KPAD, d), jnp.float32),
            pltpu.VMEM((_CHUNK, d), jnp.float32),
            pltpu.VMEM((_CHUNK, d), jnp.float32),
            pltpu.VMEM((d,), jnp.float32),
            pltpu.SemaphoreType.DMA,
        ],
    )
    out = sc(proj, idx_flat, w_bcast, feat_pad, emb.reshape(-1))

    return out[:n][None]
